# baseline (speedup 1.0000x reference)
import functools

import jax
import jax.numpy as jnp
from jax.experimental import pallas as pl
from jax.experimental.pallas import tpu as pltpu

_TEMP = 0.07
_INVT = 1.0 / _TEMP
_EMA = 0.99
_FMAX = jnp.finfo(jnp.float32).max
_FEPS = jnp.finfo(jnp.float32).eps


def _group_fold(v, qk, k, op):
    lane = jax.lax.broadcasted_iota(jnp.int32, (1, qk), 1)
    s = k // 2
    while s >= 1:
        up = pltpu.roll(v, qk - s, 1)
        dn = pltpu.roll(v, s, 1)
        other = jnp.where((lane & s) == 0, up, dn)
        v = op(v, other)
        s //= 2
    return v


def _logit_kernel(x_ref, d_ref, o_ref, *, q, k):
    qk = q * k
    x = x_ref[0]
    m_col = jnp.max(x, axis=0, keepdims=True)
    m_b = _group_fold(m_col, qk, k, jnp.maximum)
    e = jnp.exp((x - m_b) * _INVT)
    s_col = jnp.sum(e, axis=0, keepdims=True)
    s_b = _group_fold(s_col, qk, k, jnp.add)
    lse_b = m_b * _INVT + jnp.log(s_b)
    o_ref[0] = d_ref[0] * _INVT - lse_b


def _update_kernel(idx_ref, row_ref, logit_ref, mask_ref, det_ref,
                   pseudo_ref, conf_ref, newconf_ref, loss_ref,
                   num_acc, den_acc, *, b, q, k):
    i = pl.program_id(0)

    @pl.when(i == 0)
    def _init():
        num_acc[0, 0] = 0.0
        den_acc[0, 0] = 0.0

    row = row_ref[0]
    lg = logit_ref[0]
    mkf = mask_ref[0]
    mk = mkf != 0.0
    det2 = jnp.broadcast_to(det_ref[0, 0][None, :], (q, k))

    pseudo = jnp.where(mk, row, 0.0)
    pseudo_ref[0] = pseudo
    num_acc[0, 0] += jnp.sum(pseudo * lg)
    den_acc[0, 0] += jnp.sum(mkf[:, 0:1])

    cl = jnp.where(mk, lg, -_FMAX)
    mx = jnp.max(cl, axis=1, keepdims=True)
    e = jnp.exp(cl - mx)
    sm = e / jnp.sum(e, axis=1, keepdims=True)
    conf = jnp.where(mk, sm, 0.0)
    conf_ref[0] = conf

    kio = jax.lax.broadcasted_iota(jnp.int32, (q, k), 1)
    cmax = jnp.max(conf, axis=1, keepdims=True)
    amax = jnp.min(jnp.where(conf == cmax, kio, k), axis=1, keepdims=True)
    tc = jnp.where(mk, (kio == amax).astype(jnp.int32), 0)
    co = jnp.max(det2 * tc, axis=1, keepdims=True)
    tc2 = (co == det2).astype(jnp.float32)
    newconf_ref[0] = _EMA * row + (1.0 - _EMA) * tc2

    @pl.when(i == b - 1)
    def _fin():
        loss_ref[0, 0] = -num_acc[0, 0] / (den_acc[0, 0] + _FEPS)


def kernel(output, batch_index, det_labels, x_mask, confidence):
    b, b2, q, k = output.shape
    n = confidence.shape[0]
    qk = q * k

    out3 = output.reshape(b, b2, qk)
    outd = output.reshape(b * b2, 1, qk)
    logit2 = pl.pallas_call(
        functools.partial(_logit_kernel, q=q, k=k),
        grid=(b,),
        in_specs=[
            pl.BlockSpec((1, b2, qk), lambda i: (i, 0, 0)),
            pl.BlockSpec((1, 1, qk), lambda i: (i * b2 + i, 0, 0)),
        ],
        out_specs=pl.BlockSpec((1, 1, qk), lambda i: (i, 0, 0)),
        out_shape=jax.ShapeDtypeStruct((b, 1, qk), jnp.float32),
    )(out3, outd)
    logit = logit2.reshape(b, q, k)

    maskf = x_mask.astype(jnp.float32)
    det3 = det_labels.astype(jnp.int32).reshape(b, 1, k)
    idx = batch_index.astype(jnp.int32)

    grid_spec = pltpu.PrefetchScalarGridSpec(
        num_scalar_prefetch=1,
        grid=(b,),
        in_specs=[
            pl.BlockSpec((1, q, k), lambda i, idx_ref: (idx_ref[i], 0, 0)),
            pl.BlockSpec((1, q, k), lambda i, idx_ref: (i, 0, 0)),
            pl.BlockSpec((1, q, k), lambda i, idx_ref: (i, 0, 0)),
            pl.BlockSpec((1, 1, k), lambda i, idx_ref: (i, 0, 0)),
        ],
        out_specs=[
            pl.BlockSpec((1, q, k), lambda i, idx_ref: (i, 0, 0)),
            pl.BlockSpec((1, q, k), lambda i, idx_ref: (i, 0, 0)),
            pl.BlockSpec((1, q, k), lambda i, idx_ref: (idx_ref[i], 0, 0)),
            pl.BlockSpec(memory_space=pltpu.SMEM),
        ],
        scratch_shapes=[
            pltpu.SMEM((1, 1), jnp.float32),
            pltpu.SMEM((1, 1), jnp.float32),
        ],
    )
    pseudo, conf, new_conf, loss11 = pl.pallas_call(
        functools.partial(_update_kernel, b=b, q=q, k=k),
        grid_spec=grid_spec,
        out_shape=[
            jax.ShapeDtypeStruct((b, q, k), jnp.float32),
            jax.ShapeDtypeStruct((b, q, k), jnp.float32),
            jax.ShapeDtypeStruct((n, q, k), jnp.float32),
            jax.ShapeDtypeStruct((1, 1), jnp.float32),
        ],
        input_output_aliases={1: 2},
    )(idx, confidence, logit, maskf, det3)

    return (loss11[0, 0], logit, pseudo, conf, new_conf)


# speedup vs baseline: 3.2662x; 3.2662x over previous
import functools

import jax
import jax.numpy as jnp
from jax.experimental import pallas as pl
from jax.experimental.pallas import tpu as pltpu

_TEMP = 0.07
_INVT = 1.0 / _TEMP
_EMA = 0.99
_FMAX = jnp.finfo(jnp.float32).max
_FEPS = jnp.finfo(jnp.float32).eps
_LB = 128


def _logit_kernel(x_ref, o_ref, *, q, k, b2):
    i = pl.program_id(0)
    x = x_ref[0]
    mq3 = jnp.max(jnp.max(x, axis=2, keepdims=True), axis=1, keepdims=True)
    e = jnp.exp((x - mq3) * _INVT)
    sq = jnp.sum(jnp.sum(e, axis=2), axis=1, keepdims=True)
    mq2 = jnp.max(jnp.max(x, axis=2), axis=1, keepdims=True)
    lse = mq2 * _INVT + jnp.log(sq)
    li = jax.lax.broadcasted_iota(jnp.int32, (q, k, b2), 2)
    d = jnp.sum(jnp.where(li == i, x, 0.0), axis=2)
    o_ref[0] = d * _INVT - lse


def _gather_kernel(blk_ref, lane_ref, pos_ref, cblk_ref, old_ref, *, q, k):
    i = pl.program_id(0)
    cblk = cblk_ref[...]
    li = jax.lax.broadcasted_iota(jnp.int32, (q, k, _LB), 2)
    old_ref[0] = jnp.sum(jnp.where(li == lane_ref[i], cblk, 0.0), axis=2)


def _math_kernel(old_ref, logit_ref, mask_ref, det_ref,
                 pseudo_ref, conf_ref, newrows_ref, loss_ref, *, b, q, k):
    row = old_ref[...]
    lg = logit_ref[...]
    mkf = mask_ref[...]
    mk = mkf != 0.0
    det3 = jnp.broadcast_to(det_ref[...][None, :, :], (q, k, b))

    pseudo = jnp.where(mk, row, 0.0)
    pseudo_ref[...] = pseudo
    num = jnp.sum(jnp.sum(jnp.sum(pseudo * lg, axis=2), axis=1))
    phr = jnp.sum(mkf[:, 0:1, :])

    cl = jnp.where(mk, lg, -_FMAX)
    mx = jnp.max(cl, axis=1, keepdims=True)
    e = jnp.exp(cl - mx)
    sm = e / jnp.sum(e, axis=1, keepdims=True)
    conf = jnp.where(mk, sm, 0.0)
    conf_ref[...] = conf

    kio = jax.lax.broadcasted_iota(jnp.int32, (q, k, b), 1)
    cmax = jnp.max(conf, axis=1, keepdims=True)
    amax = jnp.min(jnp.where(conf == cmax, kio, k), axis=1, keepdims=True)
    tcf = jnp.where(mk, (kio == amax).astype(jnp.int32), 0)
    co = jnp.max(det3 * tcf, axis=1, keepdims=True)
    tc2 = (co == det3).astype(jnp.float32)
    newrows_ref[...] = _EMA * row + (1.0 - _EMA) * tc2
    loss_ref[0, 0] = -num / (phr + _FEPS)


def _scatter_kernel(blk_ref, lane_ref, pos_ref, newrows_ref, cblk_ref,
                    out_ref, *, b, q, k):
    i = pl.program_id(0)
    l = lane_ref[i]
    p = pos_ref[i]
    nr = newrows_ref[...]
    li_b = jax.lax.broadcasted_iota(jnp.int32, (q, k, b), 2)
    col = jnp.where(li_b == p, nr, 0.0)
    rolled = pltpu.roll(col, jnp.mod(l - p, b), 2)
    half = rolled[:, :, 0:_LB]
    li = jax.lax.broadcasted_iota(jnp.int32, (q, k, _LB), 2)
    out_ref[...] = jnp.where(li == l, half, cblk_ref[...])


def kernel(output, batch_index, det_labels, x_mask, confidence):
    b, b2, q, k = output.shape
    n = confidence.shape[0]

    out_t = jnp.transpose(output, (0, 2, 3, 1))
    logit = pl.pallas_call(
        functools.partial(_logit_kernel, q=q, k=k, b2=b2),
        grid=(b,),
        in_specs=[pl.BlockSpec((1, q, k, b2), lambda i: (i, 0, 0, 0))],
        out_specs=pl.BlockSpec((1, q, k), lambda i: (i, 0, 0)),
        out_shape=jax.ShapeDtypeStruct((b, q, k), jnp.float32),
    )(out_t)

    idx = batch_index.astype(jnp.int32)
    order = jnp.argsort(idx // _LB)
    idx_s = idx[order]
    blk_s = idx_s // _LB
    lane_s = idx_s % _LB
    pos_s = order.astype(jnp.int32)

    conf_t = jnp.transpose(confidence, (1, 2, 0))

    old_rows = pl.pallas_call(
        functools.partial(_gather_kernel, q=q, k=k),
        grid_spec=pltpu.PrefetchScalarGridSpec(
            num_scalar_prefetch=3,
            grid=(b,),
            in_specs=[
                pl.BlockSpec((q, k, _LB), lambda i, bl, la, po: (0, 0, bl[i])),
            ],
            out_specs=pl.BlockSpec((1, q, k), lambda i, bl, la, po: (po[i], 0, 0)),
        ),
        out_shape=jax.ShapeDtypeStruct((b, q, k), jnp.float32),
    )(blk_s, lane_s, pos_s, conf_t)

    old_t = jnp.transpose(old_rows, (1, 2, 0))
    logit_t = jnp.transpose(logit, (1, 2, 0))
    mask_t = jnp.transpose(x_mask, (1, 2, 0)).astype(jnp.float32)
    det_t = jnp.transpose(det_labels.astype(jnp.int32), (1, 0))

    pseudo_t, conf_out_t, newrows_t, loss11 = pl.pallas_call(
        functools.partial(_math_kernel, b=b, q=q, k=k),
        in_specs=[
            pl.BlockSpec(memory_space=pltpu.VMEM),
            pl.BlockSpec(memory_space=pltpu.VMEM),
            pl.BlockSpec(memory_space=pltpu.VMEM),
            pl.BlockSpec(memory_space=pltpu.VMEM),
        ],
        out_specs=[
            pl.BlockSpec(memory_space=pltpu.VMEM),
            pl.BlockSpec(memory_space=pltpu.VMEM),
            pl.BlockSpec(memory_space=pltpu.VMEM),
            pl.BlockSpec(memory_space=pltpu.SMEM),
        ],
        out_shape=[
            jax.ShapeDtypeStruct((q, k, b), jnp.float32),
            jax.ShapeDtypeStruct((q, k, b), jnp.float32),
            jax.ShapeDtypeStruct((q, k, b), jnp.float32),
            jax.ShapeDtypeStruct((1, 1), jnp.float32),
        ],
    )(old_t, logit_t, mask_t, det_t)

    newconf_t = pl.pallas_call(
        functools.partial(_scatter_kernel, b=b, q=q, k=k),
        grid_spec=pltpu.PrefetchScalarGridSpec(
            num_scalar_prefetch=3,
            grid=(b,),
            in_specs=[
                pl.BlockSpec(memory_space=pltpu.VMEM),
                pl.BlockSpec((q, k, _LB), lambda i, bl, la, po: (0, 0, bl[i])),
            ],
            out_specs=pl.BlockSpec((q, k, _LB), lambda i, bl, la, po: (0, 0, bl[i])),
        ),
        out_shape=jax.ShapeDtypeStruct((q, k, n), jnp.float32),
        input_output_aliases={4: 0},
    )(blk_s, lane_s, pos_s, newrows_t, conf_t)

    pseudo = jnp.transpose(pseudo_t, (2, 0, 1))
    conf_out = jnp.transpose(conf_out_t, (2, 0, 1))
    new_conf = jnp.transpose(newconf_t, (2, 0, 1))
    return (loss11[0, 0], logit, pseudo, conf_out, new_conf)


# speedup vs baseline: 4.5325x; 1.3877x over previous
import functools

import jax
import jax.numpy as jnp
from jax.experimental import pallas as pl
from jax.experimental.pallas import tpu as pltpu

_TEMP = 0.07
_INVT = 1.0 / _TEMP
_EMA = 0.99
_FMAX = jnp.finfo(jnp.float32).max
_FEPS = jnp.finfo(jnp.float32).eps
_LB = 128
_BI = 2


def _logit_kernel(x_ref, o_ref, *, q, k, b2, bi):
    pid = pl.program_id(0)
    li = jax.lax.broadcasted_iota(jnp.int32, (q, k, b2), 2)
    for j in range(bi):
        x = x_ref[j]
        mq3 = jnp.max(jnp.max(x, axis=2, keepdims=True), axis=1, keepdims=True)
        e = jnp.exp((x - mq3) * _INVT)
        sq = jnp.sum(jnp.sum(e, axis=2), axis=1, keepdims=True)
        mq2 = jnp.max(jnp.max(x, axis=2), axis=1, keepdims=True)
        lse = mq2 * _INVT + jnp.log(sq)
        d = jnp.sum(jnp.where(li == pid * bi + j, x, 0.0), axis=2)
        o_ref[j] = d * _INVT - lse


def _math_kernel(logit_ref, mask_ref, det_ref, conf_ref, tc2_ref, *, b, q, k):
    lg = logit_ref[...]
    mkf = mask_ref[...]
    mk = mkf != 0.0
    det3 = jnp.broadcast_to(det_ref[...][None, :, :], (q, k, b))

    cl = jnp.where(mk, lg, -_FMAX)
    mx = jnp.max(cl, axis=1, keepdims=True)
    e = jnp.exp(cl - mx)
    sm = e / jnp.sum(e, axis=1, keepdims=True)
    conf = jnp.where(mk, sm, 0.0)
    conf_ref[...] = conf

    kio = jax.lax.broadcasted_iota(jnp.int32, (q, k, b), 1)
    cmax = jnp.max(conf, axis=1, keepdims=True)
    amax = jnp.min(jnp.where(conf == cmax, kio, k), axis=1, keepdims=True)
    tcf = jnp.where(mk, (kio == amax).astype(jnp.int32), 0)
    co = jnp.max(det3 * tcf, axis=1, keepdims=True)
    tc2_ref[...] = (co == det3).astype(jnp.float32)


def _bank_kernel(blk_ref, lane_ref, pos_ref, fresh_ref,
                 cblk_ref, tc2_ref, logit_ref, mask_ref,
                 out_ref, pseudo_ref, loss_ref,
                 num_acc, den_acc, *, b, q, k):
    i = pl.program_id(0)
    l = lane_ref[i]
    p = pos_ref[i]

    @pl.when(i == 0)
    def _init():
        num_acc[0, 0] = 0.0
        den_acc[0, 0] = 0.0

    cblk = cblk_ref[...]
    li = jax.lax.broadcasted_iota(jnp.int32, (q, k, _LB), 2)
    oldrow = jnp.sum(jnp.where(li == l, cblk, 0.0), axis=2)
    mrow = mask_ref[0]
    lrow = logit_ref[0]
    ps = jnp.where(mrow != 0.0, oldrow, 0.0)
    pseudo_ref[0] = ps
    num_acc[0, 0] += jnp.sum(ps * lrow)
    den_acc[0, 0] += jnp.sum(mrow[:, 0:1])

    li_b = jax.lax.broadcasted_iota(jnp.int32, (q, k, b), 2)
    col = jnp.where(li_b == p, tc2_ref[...], 0.0)
    tc2l = pltpu.roll(col, jnp.mod(l - p, b), 2)[:, :, 0:_LB]

    @pl.when(fresh_ref[i] == 1)
    def _first_visit():
        out_ref[...] = cblk

    base = out_ref[...]
    out_ref[...] = jnp.where(li == l, _EMA * base + (1.0 - _EMA) * tc2l, base)

    @pl.when(i == b - 1)
    def _fin():
        loss_ref[0, 0] = -num_acc[0, 0] / (den_acc[0, 0] + _FEPS)


def kernel(output, batch_index, det_labels, x_mask, confidence):
    b, b2, q, k = output.shape
    n = confidence.shape[0]

    out_t = jnp.transpose(output, (0, 2, 3, 1))
    logit = pl.pallas_call(
        functools.partial(_logit_kernel, q=q, k=k, b2=b2, bi=_BI),
        grid=(b // _BI,),
        in_specs=[pl.BlockSpec((_BI, q, k, b2), lambda i: (i, 0, 0, 0))],
        out_specs=pl.BlockSpec((_BI, q, k), lambda i: (i, 0, 0)),
        out_shape=jax.ShapeDtypeStruct((b, q, k), jnp.float32),
    )(out_t)

    idx = batch_index.astype(jnp.int32)
    order = jnp.argsort(idx // _LB)
    idx_s = idx[order]
    blk_s = idx_s // _LB
    lane_s = idx_s % _LB
    pos_s = order.astype(jnp.int32)
    fresh_s = jnp.concatenate(
        [jnp.ones((1,), jnp.int32), (blk_s[1:] != blk_s[:-1]).astype(jnp.int32)])

    conf_t = jnp.transpose(confidence, (1, 2, 0))
    logit_t = jnp.transpose(logit, (1, 2, 0))
    mask_f = x_mask.astype(jnp.float32)
    mask_t = jnp.transpose(mask_f, (1, 2, 0))
    det_t = jnp.transpose(det_labels.astype(jnp.int32), (1, 0))

    conf_out_t, tc2_t = pl.pallas_call(
        functools.partial(_math_kernel, b=b, q=q, k=k),
        in_specs=[
            pl.BlockSpec(memory_space=pltpu.VMEM),
            pl.BlockSpec(memory_space=pltpu.VMEM),
            pl.BlockSpec(memory_space=pltpu.VMEM),
        ],
        out_specs=[
            pl.BlockSpec(memory_space=pltpu.VMEM),
            pl.BlockSpec(memory_space=pltpu.VMEM),
        ],
        out_shape=[
            jax.ShapeDtypeStruct((q, k, b), jnp.float32),
            jax.ShapeDtypeStruct((q, k, b), jnp.float32),
        ],
    )(logit_t, mask_t, det_t)

    newconf_t, pseudo, loss11 = pl.pallas_call(
        functools.partial(_bank_kernel, b=b, q=q, k=k),
        grid_spec=pltpu.PrefetchScalarGridSpec(
            num_scalar_prefetch=4,
            grid=(b,),
            in_specs=[
                pl.BlockSpec((q, k, _LB), lambda i, bl, la, po, fr: (0, 0, bl[i])),
                pl.BlockSpec(memory_space=pltpu.VMEM),
                pl.BlockSpec((1, q, k), lambda i, bl, la, po, fr: (po[i], 0, 0)),
                pl.BlockSpec((1, q, k), lambda i, bl, la, po, fr: (po[i], 0, 0)),
            ],
            out_specs=[
                pl.BlockSpec((q, k, _LB), lambda i, bl, la, po, fr: (0, 0, bl[i])),
                pl.BlockSpec((1, q, k), lambda i, bl, la, po, fr: (po[i], 0, 0)),
                pl.BlockSpec(memory_space=pltpu.SMEM),
            ],
            scratch_shapes=[
                pltpu.SMEM((1, 1), jnp.float32),
                pltpu.SMEM((1, 1), jnp.float32),
            ],
        ),
        out_shape=[
            jax.ShapeDtypeStruct((q, k, n), jnp.float32),
            jax.ShapeDtypeStruct((b, q, k), jnp.float32),
            jax.ShapeDtypeStruct((1, 1), jnp.float32),
        ],
        input_output_aliases={4: 0},
    )(blk_s, lane_s, pos_s, fresh_s, conf_t, tc2_t, logit, mask_f)

    conf_out = jnp.transpose(conf_out_t, (2, 0, 1))
    new_conf = jnp.transpose(newconf_t, (2, 0, 1))
    return (loss11[0, 0], logit, pseudo, conf_out, new_conf)


# speedup vs baseline: 4.8052x; 1.0602x over previous
import functools

import jax
import jax.numpy as jnp
from jax.experimental import pallas as pl
from jax.experimental.pallas import tpu as pltpu

_TEMP = 0.07
_INVT = 1.0 / _TEMP
_EMA = 0.99
_FMAX = jnp.finfo(jnp.float32).max
_FEPS = jnp.finfo(jnp.float32).eps
_LB = 128
_BI = 4


def _logit_kernel(x_ref, o_ref, *, q, k, b2, bi):
    pid = pl.program_id(0)
    li = jax.lax.broadcasted_iota(jnp.int32, (q, k, b2), 2)
    for j in range(bi):
        x = x_ref[j]
        mq3 = jnp.max(jnp.max(x, axis=2, keepdims=True), axis=1, keepdims=True)
        e = jnp.exp((x - mq3) * _INVT)
        sq = jnp.sum(jnp.sum(e, axis=2), axis=1, keepdims=True)
        mq2 = jnp.max(jnp.max(x, axis=2), axis=1, keepdims=True)
        lse = mq2 * _INVT + jnp.log(sq)
        d = jnp.sum(jnp.where(li == pid * bi + j, x, 0.0), axis=2)
        o_ref[j] = d * _INVT - lse


def _math_kernel(logit_ref, mask_ref, det_ref, conf_ref, tc2_ref, *, b, q, k):
    lg = logit_ref[...]
    mkf = mask_ref[...]
    mk = mkf != 0.0
    det3 = jnp.broadcast_to(det_ref[...][None, :, :], (q, k, b))

    cl = jnp.where(mk, lg, -_FMAX)
    mx = jnp.max(cl, axis=1, keepdims=True)
    e = jnp.exp(cl - mx)
    sm = e / jnp.sum(e, axis=1, keepdims=True)
    conf = jnp.where(mk, sm, 0.0)
    conf_ref[...] = conf

    kio = jax.lax.broadcasted_iota(jnp.int32, (q, k, b), 1)
    cmax = jnp.max(conf, axis=1, keepdims=True)
    amax = jnp.min(jnp.where(conf == cmax, kio, k), axis=1, keepdims=True)
    tcf = jnp.where(mk, (kio == amax).astype(jnp.int32), 0)
    co = jnp.max(det3 * tcf, axis=1, keepdims=True)
    tc2_ref[...] = (co == det3).astype(jnp.float32)


def _bank_kernel(blk_ref, lane_ref, pos_ref, fresh_ref,
                 cblk_ref, tc2_ref, logit_ref, mask_ref,
                 out_ref, pseudo_ref, loss_ref,
                 num_acc, den_acc, *, b, q, k):
    i = pl.program_id(0)
    l = lane_ref[i]
    p = pos_ref[i]

    @pl.when(i == 0)
    def _init():
        num_acc[0, 0] = 0.0
        den_acc[0, 0] = 0.0

    cblk = cblk_ref[...]
    li = jax.lax.broadcasted_iota(jnp.int32, (q, k, _LB), 2)
    oldrow = jnp.sum(jnp.where(li == l, cblk, 0.0), axis=2)
    mrow = mask_ref[0]
    lrow = logit_ref[0]
    ps = jnp.where(mrow != 0.0, oldrow, 0.0)
    pseudo_ref[0] = ps
    num_acc[0, 0] += jnp.sum(ps * lrow)
    den_acc[0, 0] += jnp.sum(mrow[:, 0:1])

    pl_ = jnp.mod(p, _LB)
    col = jnp.where(li == pl_, tc2_ref[...], 0.0)
    tc2l = pltpu.roll(col, jnp.mod(l - pl_, _LB), 2)

    @pl.when(fresh_ref[i] == 1)
    def _first_visit():
        out_ref[...] = cblk

    base = out_ref[...]
    out_ref[...] = jnp.where(li == l, _EMA * base + (1.0 - _EMA) * tc2l, base)

    @pl.when(i == b - 1)
    def _fin():
        loss_ref[0, 0] = -num_acc[0, 0] / (den_acc[0, 0] + _FEPS)


def kernel(output, batch_index, det_labels, x_mask, confidence):
    b, b2, q, k = output.shape
    n = confidence.shape[0]

    out_t = jnp.transpose(output, (0, 2, 3, 1))
    logit = pl.pallas_call(
        functools.partial(_logit_kernel, q=q, k=k, b2=b2, bi=_BI),
        grid=(b // _BI,),
        in_specs=[pl.BlockSpec((_BI, q, k, b2), lambda i: (i, 0, 0, 0))],
        out_specs=pl.BlockSpec((_BI, q, k), lambda i: (i, 0, 0)),
        out_shape=jax.ShapeDtypeStruct((b, q, k), jnp.float32),
    )(out_t)

    idx = batch_index.astype(jnp.int32)
    order = jnp.argsort(idx // _LB)
    idx_s = idx[order]
    blk_s = idx_s // _LB
    lane_s = idx_s % _LB
    pos_s = order.astype(jnp.int32)
    fresh_s = jnp.concatenate(
        [jnp.ones((1,), jnp.int32), (blk_s[1:] != blk_s[:-1]).astype(jnp.int32)])

    conf_t = jnp.transpose(confidence, (1, 2, 0))
    logit_t = jnp.transpose(logit, (1, 2, 0))
    mask_f = x_mask.astype(jnp.float32)
    mask_t = jnp.transpose(mask_f, (1, 2, 0))
    det_t = jnp.transpose(det_labels.astype(jnp.int32), (1, 0))

    conf_out_t, tc2_t = pl.pallas_call(
        functools.partial(_math_kernel, b=b, q=q, k=k),
        in_specs=[
            pl.BlockSpec(memory_space=pltpu.VMEM),
            pl.BlockSpec(memory_space=pltpu.VMEM),
            pl.BlockSpec(memory_space=pltpu.VMEM),
        ],
        out_specs=[
            pl.BlockSpec(memory_space=pltpu.VMEM),
            pl.BlockSpec(memory_space=pltpu.VMEM),
        ],
        out_shape=[
            jax.ShapeDtypeStruct((q, k, b), jnp.float32),
            jax.ShapeDtypeStruct((q, k, b), jnp.float32),
        ],
    )(logit_t, mask_t, det_t)

    newconf_t, pseudo, loss11 = pl.pallas_call(
        functools.partial(_bank_kernel, b=b, q=q, k=k),
        grid_spec=pltpu.PrefetchScalarGridSpec(
            num_scalar_prefetch=4,
            grid=(b,),
            in_specs=[
                pl.BlockSpec((q, k, _LB), lambda i, bl, la, po, fr: (0, 0, bl[i])),
                pl.BlockSpec((q, k, _LB), lambda i, bl, la, po, fr: (0, 0, po[i] // _LB)),
                pl.BlockSpec((1, q, k), lambda i, bl, la, po, fr: (po[i], 0, 0)),
                pl.BlockSpec((1, q, k), lambda i, bl, la, po, fr: (po[i], 0, 0)),
            ],
            out_specs=[
                pl.BlockSpec((q, k, _LB), lambda i, bl, la, po, fr: (0, 0, bl[i])),
                pl.BlockSpec((1, q, k), lambda i, bl, la, po, fr: (po[i], 0, 0)),
                pl.BlockSpec(memory_space=pltpu.SMEM),
            ],
            scratch_shapes=[
                pltpu.SMEM((1, 1), jnp.float32),
                pltpu.SMEM((1, 1), jnp.float32),
            ],
        ),
        out_shape=[
            jax.ShapeDtypeStruct((q, k, n), jnp.float32),
            jax.ShapeDtypeStruct((b, q, k), jnp.float32),
            jax.ShapeDtypeStruct((1, 1), jnp.float32),
        ],
        input_output_aliases={4: 0},
    )(blk_s, lane_s, pos_s, fresh_s, conf_t, tc2_t, logit, mask_f)

    conf_out = jnp.transpose(conf_out_t, (2, 0, 1))
    new_conf = jnp.transpose(newconf_t, (2, 0, 1))
    return (loss11[0, 0], logit, pseudo, conf_out, new_conf)
